# Initial kernel scaffold; baseline (speedup 1.0000x reference)
import jax
import jax.numpy as jnp
from jax.experimental import pallas as pl


def kernel(code_x, divided, neighbors, lens, user, cate, text_features, admission_times, adj, cate_adj, c_embeddings, n_embeddings, u_embeddings, cate_embeddings, Wg, bg, Wc, bc, Wz, Uz, bz, Wr, Ur, br, Wh, Uh, bh, Wout, bout, Wa, ba, va, emb_gender, emb_age, emb_cluster, bn_gamma, bn_beta, Wcls, bcls):
    raise NotImplementedError("write your pallas kernel here")


# speedup vs baseline: 12.3740x; 12.3740x over previous
import jax
import jax.numpy as jnp
from jax.experimental import pallas as pl
from jax.experimental.pallas import tpu as pltpu

CODE_NUM = 2048
CODE_SIZE = 48
GRAPH_SIZE = 32
HIDDEN = 64
ATT = 32
CATE_NUM = 128
OUT_SIZE = 2048
TEXT = 300
B = 8
T = 4
BT = B * T
NBLK = 4
BLK = CODE_NUM // NBLK


def _mm_kernel(cxa_ref, nba_ref, Ct_ref, Nt_ref, adjT_ref, out_ref, zt_ref):
    @pl.when(pl.program_id(0) == 0)
    def _build():
        Ctv = Ct_ref[...]
        Ntv = Nt_ref[...]
        for bt in range(BT):
            c = cxa_ref[bt:bt + 1, :]
            n = nba_ref[bt:bt + 1, :]
            zt_ref[bt * CODE_SIZE:(bt + 1) * CODE_SIZE, :] = Ctv * c + Ntv * n

    out_ref[...] = jnp.dot(zt_ref[...], adjT_ref[...],
                           preferred_element_type=jnp.float32)


def _graph_kernel(cx_ref, nb_ref, dv_ref, cate_ref, st_ref, Ct_ref, Nt_ref,
                  Ut_ref, CEt_ref, cadjT_ref, WgT_ref, bg_ref, WcT_ref,
                  bc_ref, xt_ref):
    cx = cx_ref[0]
    nb = nb_ref[0]
    st = st_ref[0]
    Ctv = Ct_ref[...]
    Ntv = Nt_ref[...]
    Utv = Ut_ref[...]
    WgT = WgT_ref[...]
    bg = bg_ref[...]
    no_prev = None
    for t in range(T):
        c = cx[t:t + 1, :]
        n = nb[t:t + 1, :]
        Sbt = st[t * CODE_SIZE:(t + 1) * CODE_SIZE, :]
        coT = jnp.tanh(jnp.dot(WgT, c * (Ctv + Sbt),
                               preferred_element_type=jnp.float32) + bg)
        noT = jnp.tanh(jnp.dot(WgT, n * (Ntv + Sbt),
                               preferred_element_type=jnp.float32) + bg)
        m1 = dv_ref[0, t, 0:1, :]
        m23 = dv_ref[0, t, 1:2, :] + dv_ref[0, t, 2:3, :]
        pers = jnp.sum(coT * m1, axis=1, keepdims=True)
        if t == 0:
            emer = jnp.zeros((GRAPH_SIZE, 1), jnp.float32)
        else:
            candT = m23 * no_prev
            sc = jnp.sum(candT * Utv, axis=0, keepdims=True)
            mx = jnp.max(sc)
            ex = jnp.exp(sc - mx)
            denom = jnp.sum(ex)
            emer = jnp.sum(candT * ex, axis=1, keepdims=True) / denom
        cate_row = cate_ref[0][t:t + 1, :]
        ccenT = CEt_ref[...] * cate_row
        cc2T = jnp.dot(ccenT, cadjT_ref[...],
                       preferred_element_type=jnp.float32)
        caoT = jnp.tanh(jnp.dot(WcT_ref[...], ccenT + cate_row * cc2T,
                                preferred_element_type=jnp.float32)
                        + bc_ref[...])
        pool = (jnp.sum(caoT * cate_row, axis=1, keepdims=True)
                / (jnp.sum(cate_row) + 1e-6))
        xt_ref[0, 0:GRAPH_SIZE, t:t + 1] = pers
        xt_ref[0, GRAPH_SIZE:2 * GRAPH_SIZE, t:t + 1] = emer
        xt_ref[0, 2 * GRAPH_SIZE:3 * GRAPH_SIZE, t:t + 1] = pool
        no_prev = noT


def _head_kernel(x_ref, lens_ref, user_ref, tf_ref, eg_ref, ea_ref, ec_ref,
                 Wz_ref, Uz_ref, bz_ref, Wr_ref, Ur_ref, br_ref,
                 Wh_ref, Uh_ref, bh_ref, Wout_ref, bout_ref,
                 Wa_ref, ba_ref, va_ref, gam_ref, bet_ref,
                 Wcls_ref, bcls_ref, out_ref):
    f32 = jnp.float32
    x = x_ref[...]
    h = jnp.zeros((B, HIDDEN), f32)
    Vs = []
    for t in range(T):
        xt = x[:, t, :]
        z = jax.nn.sigmoid(jnp.dot(xt, Wz_ref[...], preferred_element_type=f32)
                           + jnp.dot(h, Uz_ref[...], preferred_element_type=f32)
                           + bz_ref[...])
        r = jax.nn.sigmoid(jnp.dot(xt, Wr_ref[...], preferred_element_type=f32)
                           + jnp.dot(h, Ur_ref[...], preferred_element_type=f32)
                           + br_ref[...])
        hh = jnp.tanh(jnp.dot(xt, Wh_ref[...], preferred_element_type=f32)
                      + jnp.dot(r * h, Uh_ref[...], preferred_element_type=f32)
                      + bh_ref[...])
        h = (1.0 - z) * h + z * hh
        Vs.append(jnp.tanh(jnp.dot(h, Wout_ref[...], preferred_element_type=f32)
                           + bout_ref[...]))
    scs = []
    for t in range(T):
        u = jnp.tanh(jnp.dot(Vs[t], Wa_ref[...], preferred_element_type=f32)
                     + ba_ref[...])
        scs.append(jnp.dot(u, va_ref[...], preferred_element_type=f32))
    sc = jnp.concatenate(scs, axis=1)
    len_c = jnp.maximum(lens_ref[...], 1)
    tio = jax.lax.broadcasted_iota(jnp.int32, (B, T), 1)
    sc = jnp.where(tio < len_c, sc, -jnp.inf)
    mx = jnp.max(sc, axis=1, keepdims=True)
    ex = jnp.exp(sc - mx)
    al = ex / jnp.sum(ex, axis=1, keepdims=True)
    pooled = jnp.zeros((B, HIDDEN), f32)
    for t in range(T):
        pooled = pooled + al[:, t:t + 1] * Vs[t]
    g = user_ref[...]
    oh1 = (jax.lax.broadcasted_iota(jnp.int32, (B, 2), 1)
           == g[:, 0:1]).astype(f32)
    oh2 = (jax.lax.broadcasted_iota(jnp.int32, (B, 9), 1)
           == g[:, 1:2]).astype(f32)
    oh3 = (jax.lax.broadcasted_iota(jnp.int32, (B, 20), 1)
           == g[:, 2:3]).astype(f32)
    u1 = jnp.dot(oh1, eg_ref[...], preferred_element_type=f32)
    u2 = jnp.dot(oh2, ea_ref[...], preferred_element_type=f32)
    u3 = jnp.dot(oh3, ec_ref[...], preferred_element_type=f32)
    tsel = (tio == (len_c - 1)).astype(f32)
    text_last = jnp.zeros((B, TEXT), f32)
    for t in range(T):
        text_last = text_last + tsel[:, t:t + 1] * tf_ref[:, t, :]
    out = jnp.concatenate([u1, u2, u3, pooled, text_last], axis=1)
    mean = jnp.mean(out, axis=0, keepdims=True)
    var = jnp.mean((out - mean) ** 2, axis=0, keepdims=True)
    outn = (out - mean) / jnp.sqrt(var + 1e-5) * gam_ref[...] + bet_ref[...]
    res = jax.nn.sigmoid(jnp.dot(outn, Wcls_ref[...],
                                 preferred_element_type=f32) + bcls_ref[...])
    out_ref[...] = res


def _forward_impl(code_x, divided, neighbors, lens, user, cate, text_features,
                  adj, cate_adj, c_embeddings, n_embeddings, u_embeddings,
                  cate_embeddings, Wg, bg, Wc, bc, Wz, Uz, bz, Wr, Ur, br,
                  Wh, Uh, bh, Wout, bout, Wa, ba, va, emb_gender, emb_age,
                  emb_cluster, bn_gamma, bn_beta, Wcls, bcls, interpret):
    f32 = jnp.float32
    cxa = code_x.reshape(BT, CODE_NUM)
    nba = neighbors.reshape(BT, CODE_NUM)
    adjT = adj.T
    Ct = c_embeddings.T
    Nt = n_embeddings.T
    Ut = u_embeddings.T
    CEt = cate_embeddings.T
    cadjT = cate_adj.T
    WgT = Wg.T
    WcT = Wc.T

    ST = pl.pallas_call(
        _mm_kernel,
        grid=(NBLK,),
        in_specs=[
            pl.BlockSpec((BT, CODE_NUM), lambda j: (0, 0)),
            pl.BlockSpec((BT, CODE_NUM), lambda j: (0, 0)),
            pl.BlockSpec((CODE_SIZE, CODE_NUM), lambda j: (0, 0)),
            pl.BlockSpec((CODE_SIZE, CODE_NUM), lambda j: (0, 0)),
            pl.BlockSpec((CODE_NUM, BLK), lambda j: (0, j)),
        ],
        out_specs=pl.BlockSpec((BT * CODE_SIZE, BLK), lambda j: (0, j)),
        out_shape=jax.ShapeDtypeStruct((BT * CODE_SIZE, CODE_NUM), f32),
        scratch_shapes=[pltpu.VMEM((BT * CODE_SIZE, CODE_NUM), f32)],
        interpret=interpret,
    )(cxa, nba, Ct, Nt, adjT)

    ST3 = ST.reshape(B, T * CODE_SIZE, CODE_NUM)
    dvt = jnp.transpose(divided, (0, 1, 3, 2))

    xcols = pl.pallas_call(
        _graph_kernel,
        grid=(B,),
        in_specs=[
            pl.BlockSpec((1, T, CODE_NUM), lambda b: (b, 0, 0)),
            pl.BlockSpec((1, T, CODE_NUM), lambda b: (b, 0, 0)),
            pl.BlockSpec((1, T, 3, CODE_NUM), lambda b: (b, 0, 0, 0)),
            pl.BlockSpec((1, T, CATE_NUM), lambda b: (b, 0, 0)),
            pl.BlockSpec((1, T * CODE_SIZE, CODE_NUM), lambda b: (b, 0, 0)),
            pl.BlockSpec((CODE_SIZE, CODE_NUM), lambda b: (0, 0)),
            pl.BlockSpec((CODE_SIZE, CODE_NUM), lambda b: (0, 0)),
            pl.BlockSpec((GRAPH_SIZE, CODE_NUM), lambda b: (0, 0)),
            pl.BlockSpec((CODE_SIZE, CATE_NUM), lambda b: (0, 0)),
            pl.BlockSpec((CATE_NUM, CATE_NUM), lambda b: (0, 0)),
            pl.BlockSpec((GRAPH_SIZE, CODE_SIZE), lambda b: (0, 0)),
            pl.BlockSpec((GRAPH_SIZE, 1), lambda b: (0, 0)),
            pl.BlockSpec((GRAPH_SIZE, CODE_SIZE), lambda b: (0, 0)),
            pl.BlockSpec((GRAPH_SIZE, 1), lambda b: (0, 0)),
        ],
        out_specs=pl.BlockSpec((1, 3 * GRAPH_SIZE, T), lambda b: (b, 0, 0)),
        out_shape=jax.ShapeDtypeStruct((B, 3 * GRAPH_SIZE, T), f32),
        interpret=interpret,
    )(code_x, neighbors, dvt, cate, ST3, Ct, Nt, Ut, CEt, cadjT,
      WgT, bg.reshape(GRAPH_SIZE, 1), WcT, bc.reshape(GRAPH_SIZE, 1))

    xfeat = jnp.transpose(xcols, (0, 2, 1))

    out = pl.pallas_call(
        _head_kernel,
        out_shape=jax.ShapeDtypeStruct((B, OUT_SIZE), f32),
        interpret=interpret,
    )(xfeat, lens.reshape(B, 1).astype(jnp.int32), user.astype(jnp.int32),
      text_features, emb_gender, emb_age, emb_cluster,
      Wz, Uz, bz.reshape(1, HIDDEN), Wr, Ur, br.reshape(1, HIDDEN),
      Wh, Uh, bh.reshape(1, HIDDEN), Wout, bout.reshape(1, HIDDEN),
      Wa, ba.reshape(1, ATT), va.reshape(ATT, 1),
      bn_gamma.reshape(1, -1), bn_beta.reshape(1, -1),
      Wcls, bcls.reshape(1, OUT_SIZE))
    return out


def kernel(code_x, divided, neighbors, lens, user, cate, text_features,
           admission_times, adj, cate_adj, c_embeddings, n_embeddings,
           u_embeddings, cate_embeddings, Wg, bg, Wc, bc, Wz, Uz, bz,
           Wr, Ur, br, Wh, Uh, bh, Wout, bout, Wa, ba, va, emb_gender,
           emb_age, emb_cluster, bn_gamma, bn_beta, Wcls, bcls):
    return _forward_impl(code_x, divided, neighbors, lens, user, cate,
                         text_features, adj, cate_adj, c_embeddings,
                         n_embeddings, u_embeddings, cate_embeddings,
                         Wg, bg, Wc, bc, Wz, Uz, bz, Wr, Ur, br, Wh, Uh, bh,
                         Wout, bout, Wa, ba, va, emb_gender, emb_age,
                         emb_cluster, bn_gamma, bn_beta, Wcls, bcls,
                         interpret=False)


# speedup vs baseline: 16.8378x; 1.3607x over previous
import jax
import jax.numpy as jnp
from jax.experimental import pallas as pl
from jax.experimental.pallas import tpu as pltpu

CODE_NUM = 2048
CODE_SIZE = 48
GRAPH_SIZE = 32
HIDDEN = 64
ATT = 32
CATE_NUM = 128
OUT_SIZE = 2048
TEXT = 300
B = 8
T = 4
BT = B * T
NBLK = 4
BLK = CODE_NUM // NBLK


def _mm_kernel(cxa_ref, nba_ref, Ct_ref, Nt_ref, adjT_ref, out_ref, zt_ref):
    @pl.when(pl.program_id(0) == 0)
    def _build():
        Ctv = Ct_ref[...]
        Ntv = Nt_ref[...]
        for bt in range(BT):
            c = cxa_ref[bt:bt + 1, :]
            n = nba_ref[bt:bt + 1, :]
            zt_ref[bt * CODE_SIZE:(bt + 1) * CODE_SIZE, :] = Ctv * c + Ntv * n

    out_ref[...] = jax.lax.dot_general(
        zt_ref[...], adjT_ref[...], (((1,), (1,)), ((), ())),
        preferred_element_type=jnp.float32)


def _graph_kernel(cx_ref, nb_ref, dv_ref, cate_ref, st_ref, Ct_ref, Nt_ref,
                  Ut_ref, CEt_ref, cadjT_ref, WgT_ref, bg_ref, WcT_ref,
                  bc_ref, xt_ref):
    cx = cx_ref[0]
    nb = nb_ref[0]
    st = st_ref[0]
    Ctv = Ct_ref[...]
    Ntv = Nt_ref[...]
    Utv = Ut_ref[...]
    WgT = WgT_ref[...]
    bg = bg_ref[...]
    no_prev = None
    for t in range(T):
        c = cx[t:t + 1, :]
        n = nb[t:t + 1, :]
        Sbt = st[t * CODE_SIZE:(t + 1) * CODE_SIZE, :]
        coT = jnp.tanh(jnp.dot(WgT, c * (Ctv + Sbt),
                               preferred_element_type=jnp.float32) + bg)
        noT = jnp.tanh(jnp.dot(WgT, n * (Ntv + Sbt),
                               preferred_element_type=jnp.float32) + bg)
        m1 = dv_ref[0, t, 0:1, :]
        m23 = dv_ref[0, t, 1:2, :] + dv_ref[0, t, 2:3, :]
        pers = jnp.sum(coT * m1, axis=1, keepdims=True)
        if t == 0:
            emer = jnp.zeros((GRAPH_SIZE, 1), jnp.float32)
        else:
            candT = m23 * no_prev
            sc = jnp.sum(candT * Utv, axis=0, keepdims=True)
            mx = jnp.max(sc)
            ex = jnp.exp(sc - mx)
            denom = jnp.sum(ex)
            emer = jnp.sum(candT * ex, axis=1, keepdims=True) / denom
        cate_row = cate_ref[0][t:t + 1, :]
        ccenT = CEt_ref[...] * cate_row
        cc2T = jnp.dot(ccenT, cadjT_ref[...],
                       preferred_element_type=jnp.float32)
        caoT = jnp.tanh(jnp.dot(WcT_ref[...], ccenT + cate_row * cc2T,
                                preferred_element_type=jnp.float32)
                        + bc_ref[...])
        pool = (jnp.sum(caoT * cate_row, axis=1, keepdims=True)
                / (jnp.sum(cate_row) + 1e-6))
        xt_ref[0, 0:GRAPH_SIZE, t:t + 1] = pers
        xt_ref[0, GRAPH_SIZE:2 * GRAPH_SIZE, t:t + 1] = emer
        xt_ref[0, 2 * GRAPH_SIZE:3 * GRAPH_SIZE, t:t + 1] = pool
        no_prev = noT


def _head_kernel(x_ref, lens_ref, user_ref, tf_ref, eg_ref, ea_ref, ec_ref,
                 Wz_ref, Uz_ref, bz_ref, Wr_ref, Ur_ref, br_ref,
                 Wh_ref, Uh_ref, bh_ref, Wout_ref, bout_ref,
                 Wa_ref, ba_ref, va_ref, gam_ref, bet_ref,
                 Wcls_ref, bcls_ref, out_ref):
    f32 = jnp.float32
    x = x_ref[...]
    h = jnp.zeros((B, HIDDEN), f32)
    Vs = []
    for t in range(T):
        xt = x[:, t, :]
        z = jax.nn.sigmoid(jnp.dot(xt, Wz_ref[...], preferred_element_type=f32)
                           + jnp.dot(h, Uz_ref[...], preferred_element_type=f32)
                           + bz_ref[...])
        r = jax.nn.sigmoid(jnp.dot(xt, Wr_ref[...], preferred_element_type=f32)
                           + jnp.dot(h, Ur_ref[...], preferred_element_type=f32)
                           + br_ref[...])
        hh = jnp.tanh(jnp.dot(xt, Wh_ref[...], preferred_element_type=f32)
                      + jnp.dot(r * h, Uh_ref[...], preferred_element_type=f32)
                      + bh_ref[...])
        h = (1.0 - z) * h + z * hh
        Vs.append(jnp.tanh(jnp.dot(h, Wout_ref[...], preferred_element_type=f32)
                           + bout_ref[...]))
    scs = []
    for t in range(T):
        u = jnp.tanh(jnp.dot(Vs[t], Wa_ref[...], preferred_element_type=f32)
                     + ba_ref[...])
        scs.append(jnp.dot(u, va_ref[...], preferred_element_type=f32))
    sc = jnp.concatenate(scs, axis=1)
    len_c = jnp.maximum(lens_ref[...], 1)
    tio = jax.lax.broadcasted_iota(jnp.int32, (B, T), 1)
    sc = jnp.where(tio < len_c, sc, -jnp.inf)
    mx = jnp.max(sc, axis=1, keepdims=True)
    ex = jnp.exp(sc - mx)
    al = ex / jnp.sum(ex, axis=1, keepdims=True)
    pooled = jnp.zeros((B, HIDDEN), f32)
    for t in range(T):
        pooled = pooled + al[:, t:t + 1] * Vs[t]
    g = user_ref[...]
    oh1 = (jax.lax.broadcasted_iota(jnp.int32, (B, 2), 1)
           == g[:, 0:1]).astype(f32)
    oh2 = (jax.lax.broadcasted_iota(jnp.int32, (B, 9), 1)
           == g[:, 1:2]).astype(f32)
    oh3 = (jax.lax.broadcasted_iota(jnp.int32, (B, 20), 1)
           == g[:, 2:3]).astype(f32)
    u1 = jnp.dot(oh1, eg_ref[...], preferred_element_type=f32)
    u2 = jnp.dot(oh2, ea_ref[...], preferred_element_type=f32)
    u3 = jnp.dot(oh3, ec_ref[...], preferred_element_type=f32)
    tsel = (tio == (len_c - 1)).astype(f32)
    text_last = jnp.zeros((B, TEXT), f32)
    for t in range(T):
        text_last = text_last + tsel[:, t:t + 1] * tf_ref[:, t, :]
    out = jnp.concatenate([u1, u2, u3, pooled, text_last], axis=1)
    mean = jnp.mean(out, axis=0, keepdims=True)
    var = jnp.mean((out - mean) ** 2, axis=0, keepdims=True)
    outn = (out - mean) / jnp.sqrt(var + 1e-5) * gam_ref[...] + bet_ref[...]
    res = jax.nn.sigmoid(jnp.dot(outn, Wcls_ref[...],
                                 preferred_element_type=f32) + bcls_ref[...])
    out_ref[...] = res


def _forward_impl(code_x, divided, neighbors, lens, user, cate, text_features,
                  adj, cate_adj, c_embeddings, n_embeddings, u_embeddings,
                  cate_embeddings, Wg, bg, Wc, bc, Wz, Uz, bz, Wr, Ur, br,
                  Wh, Uh, bh, Wout, bout, Wa, ba, va, emb_gender, emb_age,
                  emb_cluster, bn_gamma, bn_beta, Wcls, bcls, interpret):
    f32 = jnp.float32
    cxa = code_x.reshape(BT, CODE_NUM)
    nba = neighbors.reshape(BT, CODE_NUM)
    Ct = c_embeddings.T
    Nt = n_embeddings.T
    Ut = u_embeddings.T
    CEt = cate_embeddings.T
    cadjT = cate_adj.T
    WgT = Wg.T
    WcT = Wc.T

    ST = pl.pallas_call(
        _mm_kernel,
        grid=(NBLK,),
        in_specs=[
            pl.BlockSpec((BT, CODE_NUM), lambda j: (0, 0)),
            pl.BlockSpec((BT, CODE_NUM), lambda j: (0, 0)),
            pl.BlockSpec((CODE_SIZE, CODE_NUM), lambda j: (0, 0)),
            pl.BlockSpec((CODE_SIZE, CODE_NUM), lambda j: (0, 0)),
            pl.BlockSpec((BLK, CODE_NUM), lambda j: (j, 0)),
        ],
        out_specs=pl.BlockSpec((BT * CODE_SIZE, BLK), lambda j: (0, j)),
        out_shape=jax.ShapeDtypeStruct((BT * CODE_SIZE, CODE_NUM), f32),
        scratch_shapes=[pltpu.VMEM((BT * CODE_SIZE, CODE_NUM), f32)],
        interpret=interpret,
    )(cxa, nba, Ct, Nt, adj)

    ST3 = ST.reshape(B, T * CODE_SIZE, CODE_NUM)
    dvt = jnp.transpose(divided, (0, 1, 3, 2))

    xcols = pl.pallas_call(
        _graph_kernel,
        grid=(B,),
        in_specs=[
            pl.BlockSpec((1, T, CODE_NUM), lambda b: (b, 0, 0)),
            pl.BlockSpec((1, T, CODE_NUM), lambda b: (b, 0, 0)),
            pl.BlockSpec((1, T, 3, CODE_NUM), lambda b: (b, 0, 0, 0)),
            pl.BlockSpec((1, T, CATE_NUM), lambda b: (b, 0, 0)),
            pl.BlockSpec((1, T * CODE_SIZE, CODE_NUM), lambda b: (b, 0, 0)),
            pl.BlockSpec((CODE_SIZE, CODE_NUM), lambda b: (0, 0)),
            pl.BlockSpec((CODE_SIZE, CODE_NUM), lambda b: (0, 0)),
            pl.BlockSpec((GRAPH_SIZE, CODE_NUM), lambda b: (0, 0)),
            pl.BlockSpec((CODE_SIZE, CATE_NUM), lambda b: (0, 0)),
            pl.BlockSpec((CATE_NUM, CATE_NUM), lambda b: (0, 0)),
            pl.BlockSpec((GRAPH_SIZE, CODE_SIZE), lambda b: (0, 0)),
            pl.BlockSpec((GRAPH_SIZE, 1), lambda b: (0, 0)),
            pl.BlockSpec((GRAPH_SIZE, CODE_SIZE), lambda b: (0, 0)),
            pl.BlockSpec((GRAPH_SIZE, 1), lambda b: (0, 0)),
        ],
        out_specs=pl.BlockSpec((1, 3 * GRAPH_SIZE, T), lambda b: (b, 0, 0)),
        out_shape=jax.ShapeDtypeStruct((B, 3 * GRAPH_SIZE, T), f32),
        interpret=interpret,
    )(code_x, neighbors, dvt, cate, ST3, Ct, Nt, Ut, CEt, cadjT,
      WgT, bg.reshape(GRAPH_SIZE, 1), WcT, bc.reshape(GRAPH_SIZE, 1))

    xfeat = jnp.transpose(xcols, (0, 2, 1))

    out = pl.pallas_call(
        _head_kernel,
        out_shape=jax.ShapeDtypeStruct((B, OUT_SIZE), f32),
        interpret=interpret,
    )(xfeat, lens.reshape(B, 1).astype(jnp.int32), user.astype(jnp.int32),
      text_features, emb_gender, emb_age, emb_cluster,
      Wz, Uz, bz.reshape(1, HIDDEN), Wr, Ur, br.reshape(1, HIDDEN),
      Wh, Uh, bh.reshape(1, HIDDEN), Wout, bout.reshape(1, HIDDEN),
      Wa, ba.reshape(1, ATT), va.reshape(ATT, 1),
      bn_gamma.reshape(1, -1), bn_beta.reshape(1, -1),
      Wcls, bcls.reshape(1, OUT_SIZE))
    return out


def kernel(code_x, divided, neighbors, lens, user, cate, text_features,
           admission_times, adj, cate_adj, c_embeddings, n_embeddings,
           u_embeddings, cate_embeddings, Wg, bg, Wc, bc, Wz, Uz, bz,
           Wr, Ur, br, Wh, Uh, bh, Wout, bout, Wa, ba, va, emb_gender,
           emb_age, emb_cluster, bn_gamma, bn_beta, Wcls, bcls):
    return _forward_impl(code_x, divided, neighbors, lens, user, cate,
                         text_features, adj, cate_adj, c_embeddings,
                         n_embeddings, u_embeddings, cate_embeddings,
                         Wg, bg, Wc, bc, Wz, Uz, bz, Wr, Ur, br, Wh, Uh, bh,
                         Wout, bout, Wa, ba, va, emb_gender, emb_age,
                         emb_cluster, bn_gamma, bn_beta, Wcls, bcls,
                         interpret=False)


# speedup vs baseline: 17.2336x; 1.0235x over previous
import jax
import jax.numpy as jnp
from jax.experimental import pallas as pl
from jax.experimental.pallas import tpu as pltpu

CODE_NUM = 2048
CODE_SIZE = 48
GRAPH_SIZE = 32
HIDDEN = 64
ATT = 32
CATE_NUM = 128
OUT_SIZE = 2048
TEXT = 300
B = 8
T = 4
BT = B * T
NBLK = 4
BLK = CODE_NUM // NBLK


def _mm_kernel(cxa_ref, nba_ref, Ct_ref, Nt_ref, adjT_ref, out_ref, zt_ref):
    @pl.when(pl.program_id(0) == 0)
    def _build():
        Ctv = Ct_ref[...]
        Ntv = Nt_ref[...]
        for bt in range(BT):
            c = cxa_ref[bt:bt + 1, :]
            n = nba_ref[bt:bt + 1, :]
            zt_ref[bt * CODE_SIZE:(bt + 1) * CODE_SIZE, :] = (
                Ctv * c + Ntv * n).astype(jnp.bfloat16)

    out_ref[...] = jax.lax.dot_general(
        zt_ref[...], adjT_ref[...].astype(jnp.bfloat16), (((1,), (1,)), ((), ())),
        preferred_element_type=jnp.float32).astype(jnp.bfloat16)


def _graph_kernel(cx_ref, nb_ref, dv_ref, cate_ref, st_ref, Ct_ref, Nt_ref,
                  Ut_ref, CEt_ref, cadjT_ref, WgT_ref, bg_ref, WcT_ref,
                  bc_ref, xt_ref):
    cx = cx_ref[0]
    nb = nb_ref[0]
    st = st_ref[0]
    Ctv = Ct_ref[...]
    Ntv = Nt_ref[...]
    Utv = Ut_ref[...]
    WgT = WgT_ref[...]
    bg = bg_ref[...]
    no_prev = None
    for t in range(T):
        c = cx[t:t + 1, :]
        n = nb[t:t + 1, :]
        Sbt = st[t * CODE_SIZE:(t + 1) * CODE_SIZE, :].astype(jnp.float32)
        coT = jnp.tanh(jnp.dot(WgT, c * (Ctv + Sbt),
                               preferred_element_type=jnp.float32) + bg)
        noT = jnp.tanh(jnp.dot(WgT, n * (Ntv + Sbt),
                               preferred_element_type=jnp.float32) + bg)
        m1 = dv_ref[0, t, 0:1, :]
        m23 = dv_ref[0, t, 1:2, :] + dv_ref[0, t, 2:3, :]
        pers = jnp.sum(coT * m1, axis=1, keepdims=True)
        if t == 0:
            emer = jnp.zeros((GRAPH_SIZE, 1), jnp.float32)
        else:
            candT = m23 * no_prev
            sc = jnp.sum(candT * Utv, axis=0, keepdims=True)
            mx = jnp.max(sc)
            ex = jnp.exp(sc - mx)
            denom = jnp.sum(ex)
            emer = jnp.sum(candT * ex, axis=1, keepdims=True) / denom
        cate_row = cate_ref[0][t:t + 1, :]
        ccenT = CEt_ref[...] * cate_row
        cc2T = jnp.dot(ccenT, cadjT_ref[...],
                       preferred_element_type=jnp.float32)
        caoT = jnp.tanh(jnp.dot(WcT_ref[...], ccenT + cate_row * cc2T,
                                preferred_element_type=jnp.float32)
                        + bc_ref[...])
        pool = (jnp.sum(caoT * cate_row, axis=1, keepdims=True)
                / (jnp.sum(cate_row) + 1e-6))
        xt_ref[0, 0:GRAPH_SIZE, t:t + 1] = pers
        xt_ref[0, GRAPH_SIZE:2 * GRAPH_SIZE, t:t + 1] = emer
        xt_ref[0, 2 * GRAPH_SIZE:3 * GRAPH_SIZE, t:t + 1] = pool
        no_prev = noT


def _head_kernel(x_ref, lens_ref, user_ref, tf_ref, eg_ref, ea_ref, ec_ref,
                 Wz_ref, Uz_ref, bz_ref, Wr_ref, Ur_ref, br_ref,
                 Wh_ref, Uh_ref, bh_ref, Wout_ref, bout_ref,
                 Wa_ref, ba_ref, va_ref, gam_ref, bet_ref,
                 Wcls_ref, bcls_ref, out_ref):
    f32 = jnp.float32
    x = x_ref[...]
    h = jnp.zeros((B, HIDDEN), f32)
    Vs = []
    for t in range(T):
        xt = x[:, t, :]
        z = jax.nn.sigmoid(jnp.dot(xt, Wz_ref[...], preferred_element_type=f32)
                           + jnp.dot(h, Uz_ref[...], preferred_element_type=f32)
                           + bz_ref[...])
        r = jax.nn.sigmoid(jnp.dot(xt, Wr_ref[...], preferred_element_type=f32)
                           + jnp.dot(h, Ur_ref[...], preferred_element_type=f32)
                           + br_ref[...])
        hh = jnp.tanh(jnp.dot(xt, Wh_ref[...], preferred_element_type=f32)
                      + jnp.dot(r * h, Uh_ref[...], preferred_element_type=f32)
                      + bh_ref[...])
        h = (1.0 - z) * h + z * hh
        Vs.append(jnp.tanh(jnp.dot(h, Wout_ref[...], preferred_element_type=f32)
                           + bout_ref[...]))
    scs = []
    for t in range(T):
        u = jnp.tanh(jnp.dot(Vs[t], Wa_ref[...], preferred_element_type=f32)
                     + ba_ref[...])
        scs.append(jnp.dot(u, va_ref[...], preferred_element_type=f32))
    sc = jnp.concatenate(scs, axis=1)
    len_c = jnp.maximum(lens_ref[...], 1)
    tio = jax.lax.broadcasted_iota(jnp.int32, (B, T), 1)
    sc = jnp.where(tio < len_c, sc, -jnp.inf)
    mx = jnp.max(sc, axis=1, keepdims=True)
    ex = jnp.exp(sc - mx)
    al = ex / jnp.sum(ex, axis=1, keepdims=True)
    pooled = jnp.zeros((B, HIDDEN), f32)
    for t in range(T):
        pooled = pooled + al[:, t:t + 1] * Vs[t]
    g = user_ref[...]
    oh1 = (jax.lax.broadcasted_iota(jnp.int32, (B, 2), 1)
           == g[:, 0:1]).astype(f32)
    oh2 = (jax.lax.broadcasted_iota(jnp.int32, (B, 9), 1)
           == g[:, 1:2]).astype(f32)
    oh3 = (jax.lax.broadcasted_iota(jnp.int32, (B, 20), 1)
           == g[:, 2:3]).astype(f32)
    u1 = jnp.dot(oh1, eg_ref[...], preferred_element_type=f32)
    u2 = jnp.dot(oh2, ea_ref[...], preferred_element_type=f32)
    u3 = jnp.dot(oh3, ec_ref[...], preferred_element_type=f32)
    tsel = (tio == (len_c - 1)).astype(f32)
    text_last = jnp.zeros((B, TEXT), f32)
    for t in range(T):
        text_last = text_last + tsel[:, t:t + 1] * tf_ref[:, t, :]
    out = jnp.concatenate([u1, u2, u3, pooled, text_last], axis=1)
    mean = jnp.mean(out, axis=0, keepdims=True)
    var = jnp.mean((out - mean) ** 2, axis=0, keepdims=True)
    outn = (out - mean) / jnp.sqrt(var + 1e-5) * gam_ref[...] + bet_ref[...]
    res = jax.nn.sigmoid(jnp.dot(outn, Wcls_ref[...],
                                 preferred_element_type=f32) + bcls_ref[...])
    out_ref[...] = res


def _forward_impl(code_x, divided, neighbors, lens, user, cate, text_features,
                  adj, cate_adj, c_embeddings, n_embeddings, u_embeddings,
                  cate_embeddings, Wg, bg, Wc, bc, Wz, Uz, bz, Wr, Ur, br,
                  Wh, Uh, bh, Wout, bout, Wa, ba, va, emb_gender, emb_age,
                  emb_cluster, bn_gamma, bn_beta, Wcls, bcls, interpret):
    f32 = jnp.float32
    cxa = code_x.reshape(BT, CODE_NUM)
    nba = neighbors.reshape(BT, CODE_NUM)
    Ct = c_embeddings.T
    Nt = n_embeddings.T
    Ut = u_embeddings.T
    CEt = cate_embeddings.T
    cadjT = cate_adj.T
    WgT = Wg.T
    WcT = Wc.T

    ST = pl.pallas_call(
        _mm_kernel,
        grid=(NBLK,),
        in_specs=[
            pl.BlockSpec((BT, CODE_NUM), lambda j: (0, 0)),
            pl.BlockSpec((BT, CODE_NUM), lambda j: (0, 0)),
            pl.BlockSpec((CODE_SIZE, CODE_NUM), lambda j: (0, 0)),
            pl.BlockSpec((CODE_SIZE, CODE_NUM), lambda j: (0, 0)),
            pl.BlockSpec((BLK, CODE_NUM), lambda j: (j, 0)),
        ],
        out_specs=pl.BlockSpec((BT * CODE_SIZE, BLK), lambda j: (0, j)),
        out_shape=jax.ShapeDtypeStruct((BT * CODE_SIZE, CODE_NUM), jnp.bfloat16),
        scratch_shapes=[pltpu.VMEM((BT * CODE_SIZE, CODE_NUM), jnp.bfloat16)],
        interpret=interpret,
    )(cxa, nba, Ct, Nt, adj)

    ST3 = ST.reshape(B, T * CODE_SIZE, CODE_NUM)
    dvt = jnp.transpose(divided, (0, 1, 3, 2))

    xcols = pl.pallas_call(
        _graph_kernel,
        grid=(B,),
        in_specs=[
            pl.BlockSpec((1, T, CODE_NUM), lambda b: (b, 0, 0)),
            pl.BlockSpec((1, T, CODE_NUM), lambda b: (b, 0, 0)),
            pl.BlockSpec((1, T, 3, CODE_NUM), lambda b: (b, 0, 0, 0)),
            pl.BlockSpec((1, T, CATE_NUM), lambda b: (b, 0, 0)),
            pl.BlockSpec((1, T * CODE_SIZE, CODE_NUM), lambda b: (b, 0, 0)),
            pl.BlockSpec((CODE_SIZE, CODE_NUM), lambda b: (0, 0)),
            pl.BlockSpec((CODE_SIZE, CODE_NUM), lambda b: (0, 0)),
            pl.BlockSpec((GRAPH_SIZE, CODE_NUM), lambda b: (0, 0)),
            pl.BlockSpec((CODE_SIZE, CATE_NUM), lambda b: (0, 0)),
            pl.BlockSpec((CATE_NUM, CATE_NUM), lambda b: (0, 0)),
            pl.BlockSpec((GRAPH_SIZE, CODE_SIZE), lambda b: (0, 0)),
            pl.BlockSpec((GRAPH_SIZE, 1), lambda b: (0, 0)),
            pl.BlockSpec((GRAPH_SIZE, CODE_SIZE), lambda b: (0, 0)),
            pl.BlockSpec((GRAPH_SIZE, 1), lambda b: (0, 0)),
        ],
        out_specs=pl.BlockSpec((1, 3 * GRAPH_SIZE, T), lambda b: (b, 0, 0)),
        out_shape=jax.ShapeDtypeStruct((B, 3 * GRAPH_SIZE, T), f32),
        interpret=interpret,
    )(code_x, neighbors, dvt, cate, ST3, Ct, Nt, Ut, CEt, cadjT,
      WgT, bg.reshape(GRAPH_SIZE, 1), WcT, bc.reshape(GRAPH_SIZE, 1))

    xfeat = jnp.transpose(xcols, (0, 2, 1))

    out = pl.pallas_call(
        _head_kernel,
        out_shape=jax.ShapeDtypeStruct((B, OUT_SIZE), f32),
        interpret=interpret,
    )(xfeat, lens.reshape(B, 1).astype(jnp.int32), user.astype(jnp.int32),
      text_features, emb_gender, emb_age, emb_cluster,
      Wz, Uz, bz.reshape(1, HIDDEN), Wr, Ur, br.reshape(1, HIDDEN),
      Wh, Uh, bh.reshape(1, HIDDEN), Wout, bout.reshape(1, HIDDEN),
      Wa, ba.reshape(1, ATT), va.reshape(ATT, 1),
      bn_gamma.reshape(1, -1), bn_beta.reshape(1, -1),
      Wcls, bcls.reshape(1, OUT_SIZE))
    return out


def kernel(code_x, divided, neighbors, lens, user, cate, text_features,
           admission_times, adj, cate_adj, c_embeddings, n_embeddings,
           u_embeddings, cate_embeddings, Wg, bg, Wc, bc, Wz, Uz, bz,
           Wr, Ur, br, Wh, Uh, bh, Wout, bout, Wa, ba, va, emb_gender,
           emb_age, emb_cluster, bn_gamma, bn_beta, Wcls, bcls):
    return _forward_impl(code_x, divided, neighbors, lens, user, cate,
                         text_features, adj, cate_adj, c_embeddings,
                         n_embeddings, u_embeddings, cate_embeddings,
                         Wg, bg, Wc, bc, Wz, Uz, bz, Wr, Ur, br, Wh, Uh, bh,
                         Wout, bout, Wa, ba, va, emb_gender, emb_age,
                         emb_cluster, bn_gamma, bn_beta, Wcls, bcls,
                         interpret=False)
